# baseline (speedup 1.0000x reference)
import functools

import jax
import jax.numpy as jnp
from jax import lax
from jax.experimental import pallas as pl
from jax.experimental.pallas import tpu as pltpu
from jax.experimental.pallas import tpu_sc as plsc


def _sc_static_body(sidx_hbm, stab_hbm, out_hbm, idx_v, rows_v, sem):
    wid = lax.axis_index("s") * 2 + lax.axis_index("c")
    nb = idx_v.shape[0]
    i = wid // 8
    base = (wid % 8) * nb
    pltpu.sync_copy(sidx_hbm.at[i, pl.ds(base, nb)], idx_v)
    pltpu.async_copy(stab_hbm.at[idx_v], rows_v, sem).wait()
    pltpu.sync_copy(rows_v, out_hbm.at[pl.ds(base, nb), i])


def _sc_static(sidx_b, stab2, B, n_static, L):
    mesh = plsc.VectorSubcoreMesh(core_axis_name="c", subcore_axis_name="s")
    nb = B // 8
    f = pl.kernel(
        _sc_static_body,
        mesh=mesh,
        out_type=jax.ShapeDtypeStruct((B, n_static, L), jnp.float32),
        scratch_types=[
            pltpu.VMEM((nb,), jnp.int32),
            pltpu.VMEM((nb, L), jnp.float32),
            pltpu.SemaphoreType.DMA,
        ],
    )
    return f(sidx_b, stab2)


def _known_body(x_ref, cat_ref, tab_ref, w_ref, b_ref, xo_ref, wo_ref,
                bo_ref, out_ref, obs_ref, *, vocab):
    T, n_real, BB = x_ref.shape[0], x_ref.shape[1], x_ref.shape[2]
    n_cat = cat_ref.shape[-1]
    L = tab_ref.shape[-1]
    out_ref[:, :n_real, :, :] = (x_ref[...] * w_ref[...]
                                 + b_ref[...])
    iota_v = jax.lax.broadcasted_iota(jnp.int32, (T, BB, vocab), 2)
    for j in range(n_cat):
        idx = cat_ref[:, :, j:j + 1]
        onehot = (idx == iota_v).astype(jnp.float32).reshape(T * BB, vocab)
        g = jnp.dot(onehot, tab_ref[j], preferred_element_type=jnp.float32)
        out_ref[:, n_real + j, :, :] = g.reshape(T, BB, L)
    obs_ref[...] = (xo_ref[...] * wo_ref[...][None, None, :, :]
                    + bo_ref[...][None, None, :, :])


def _obs_body(x_ref, w_ref, b_ref, out_ref):
    out_ref[...] = (x_ref[...] * w_ref[...][None, None, :, :]
                    + b_ref[...][None, None, :, :])


def kernel(static, known_real, known_categorical, observed, static_tables,
           known_cat_tables, real_W, real_b, obs_W, obs_b):
    B, T, n_real = known_real.shape
    n_obs = observed.shape[-1]
    n_cat = known_categorical.shape[-1]
    n_static = static_tables.shape[0]
    vocab, L = static_tables.shape[1], static_tables.shape[2]
    n_known = n_real + n_cat

    xk_t = known_real.transpose(1, 2, 0)[..., None]
    cat_t = known_categorical.transpose(1, 0, 2).astype(jnp.int32)
    xo4 = observed[..., None]
    w4 = real_W[None, :, None, :]
    b4 = real_b[None, :, None, :]

    full = lambda shape: pl.BlockSpec(shape, lambda *a: (0,) * len(shape))
    BB = 16
    known_p, obs_p = pl.pallas_call(
        functools.partial(_known_body, vocab=vocab),
        grid=(B // BB,),
        in_specs=[
            pl.BlockSpec((T, n_real, BB, 1), lambda r: (0, 0, r, 0)),
            pl.BlockSpec((T, BB, n_cat), lambda r: (0, r, 0)),
            full((n_cat, vocab, L)),
            full((1, n_real, 1, L)),
            full((1, n_real, 1, L)),
            pl.BlockSpec((BB, T, n_obs, 1), lambda r: (r, 0, 0, 0)),
            full((n_obs, L)),
            full((n_obs, L)),
        ],
        out_specs=[
            pl.BlockSpec((T, n_known, BB, L), lambda r: (0, 0, r, 0)),
            pl.BlockSpec((BB, T, n_obs, L), lambda r: (r, 0, 0, 0)),
        ],
        out_shape=[
            jax.ShapeDtypeStruct((T, n_known, B, L), jnp.float32),
            jax.ShapeDtypeStruct((B, T, n_obs, L), jnp.float32),
        ],
    )(xk_t, cat_t, known_cat_tables, w4, b4, xo4, obs_W, obs_b)

    sidx_b = (static[:, 0, :].astype(jnp.int32).T
              + jnp.arange(n_static, dtype=jnp.int32)[:, None] * vocab)
    stab2 = static_tables.reshape(n_static * vocab, L)
    static_embs = _sc_static(sidx_b, stab2, B, n_static, L)

    return (static_embs,
            jnp.transpose(known_p, (2, 0, 3, 1)),
            jnp.swapaxes(obs_p, 2, 3))


# speedup vs baseline: 1.0002x; 1.0002x over previous
import functools

import jax
import jax.numpy as jnp
from jax import lax
from jax.experimental import pallas as pl
from jax.experimental.pallas import tpu as pltpu
from jax.experimental.pallas import tpu_sc as plsc


def _sc_static_body(sidx_hbm, stab_hbm, out_hbm, idx_v, rows_v, sem):
    wid = lax.axis_index("s") * 2 + lax.axis_index("c")
    nb = idx_v.shape[0]
    i = wid // 8
    base = (wid % 8) * nb
    pltpu.sync_copy(sidx_hbm.at[i, pl.ds(base, nb)], idx_v)
    pltpu.async_copy(stab_hbm.at[idx_v], rows_v, sem).wait()
    pltpu.sync_copy(rows_v, out_hbm.at[pl.ds(base, nb), i])


def _sc_static(sidx_b, stab2, B, n_static, L):
    mesh = plsc.VectorSubcoreMesh(core_axis_name="c", subcore_axis_name="s")
    nb = B // 8
    f = pl.kernel(
        _sc_static_body,
        mesh=mesh,
        out_type=jax.ShapeDtypeStruct((B, n_static, L), jnp.float32),
        scratch_types=[
            pltpu.VMEM((nb,), jnp.int32),
            pltpu.VMEM((nb, L), jnp.float32),
            pltpu.SemaphoreType.DMA,
        ],
    )
    return f(sidx_b, stab2)


def _known_obs_body(x_ref, cat_ref, tab_ref, w_ref, b_ref, xo_ref, wo_ref,
                    bo_ref, out_ref, obs_ref, *, vocab):
    T, n_real, BB = x_ref.shape[0], x_ref.shape[1], x_ref.shape[2]
    n_cat = cat_ref.shape[-1]
    L = tab_ref.shape[-1]
    out_ref[:, :n_real, :, :] = (x_ref[...] * w_ref[...]
                                 + b_ref[...])
    iota_v = jax.lax.broadcasted_iota(jnp.int32, (T, BB, vocab), 2)
    for j in range(n_cat):
        idx = cat_ref[:, :, j:j + 1]
        onehot = (idx == iota_v).astype(jnp.float32).reshape(T * BB, vocab)
        g = jnp.dot(onehot, tab_ref[j], preferred_element_type=jnp.float32)
        out_ref[:, n_real + j, :, :] = g.reshape(T, BB, L)
    obs_ref[...] = (xo_ref[...] * wo_ref[...][None, None, :, :]
                    + bo_ref[...][None, None, :, :])


def kernel(static, known_real, known_categorical, observed, static_tables,
           known_cat_tables, real_W, real_b, obs_W, obs_b):
    B, T, n_real = known_real.shape
    n_obs = observed.shape[-1]
    n_cat = known_categorical.shape[-1]
    n_static = static_tables.shape[0]
    vocab, L = static_tables.shape[1], static_tables.shape[2]
    n_known = n_real + n_cat

    sidx_b = (static[:, 0, :].astype(jnp.int32).T
              + jnp.arange(n_static, dtype=jnp.int32)[:, None] * vocab)
    stab2 = static_tables.reshape(n_static * vocab, L)
    static_embs = _sc_static(sidx_b, stab2, B, n_static, L)

    xk_t = known_real.transpose(1, 2, 0)[..., None]
    cat_t = known_categorical.transpose(1, 0, 2).astype(jnp.int32)
    xo4 = observed[..., None]
    w4 = real_W[None, :, None, :]
    b4 = real_b[None, :, None, :]

    full = lambda shape: pl.BlockSpec(shape, lambda *a: (0,) * len(shape))
    BB = 16
    known_p, obs_p = pl.pallas_call(
        functools.partial(_known_obs_body, vocab=vocab),
        grid=(B // BB,),
        in_specs=[
            pl.BlockSpec((T, n_real, BB, 1), lambda r: (0, 0, r, 0)),
            pl.BlockSpec((T, BB, n_cat), lambda r: (0, r, 0)),
            full((n_cat, vocab, L)),
            full((1, n_real, 1, L)),
            full((1, n_real, 1, L)),
            pl.BlockSpec((BB, T, n_obs, 1), lambda r: (r, 0, 0, 0)),
            full((n_obs, L)),
            full((n_obs, L)),
        ],
        out_specs=[
            pl.BlockSpec((T, n_known, BB, L), lambda r: (0, 0, r, 0)),
            pl.BlockSpec((BB, T, n_obs, L), lambda r: (r, 0, 0, 0)),
        ],
        out_shape=[
            jax.ShapeDtypeStruct((T, n_known, B, L), jnp.float32),
            jax.ShapeDtypeStruct((B, T, n_obs, L), jnp.float32),
        ],
    )(xk_t, cat_t, known_cat_tables, w4, b4, xo4, obs_W, obs_b)

    return (static_embs,
            jnp.transpose(known_p, (2, 0, 3, 1)),
            jnp.swapaxes(obs_p, 2, 3))
